# baseline (speedup 1.0000x reference)
import functools
import jax
import jax.numpy as jnp
from jax import lax
from jax.experimental import pallas as pl
from jax.experimental.pallas import tpu as pltpu
from jax.experimental.pallas import tpu_sc as plsc

N = 10000
NPAD = 10240
E = 320000
F = 128
C = 10
NC = 2
NS = 16
NW = NC * NS
EPT = E // NW
CK = 125
NCHUNK = EPT // CK
EPT2 = E // NS
NCHUNK2 = EPT2 // CK
FH = F // NC
RPT = NPAD // NS
BLK = 2048
GRID = NPAD // BLK


def _mesh():
    return plsc.VectorSubcoreMesh(core_axis_name="c", subcore_axis_name="s")


def _deg_body(dst_hbm, ew_hbm, out0_hbm, out1_hbm, dst_v, ew_v, zb_v, acc):
    c = lax.axis_index("c")
    s = lax.axis_index("s")
    wid = s * NC + c

    def zb(i, _):
        zb_v[pl.ds(i * 16, 16)] = jnp.zeros((16,), jnp.float32)
        return 0

    lax.fori_loop(0, RPT // 16, zb, 0)
    pltpu.sync_copy(zb_v, acc.at[pl.ds(s * RPT, RPT)])
    plsc.subcore_barrier()

    pltpu.sync_copy(dst_hbm.at[pl.ds(wid * NCHUNK, NCHUNK)], dst_v)
    pltpu.sync_copy(ew_hbm.at[pl.ds(wid * NCHUNK, NCHUNK)], ew_v)

    def body(j, _):
        pltpu.sync_copy(ew_v.at[j], acc.at[dst_v.at[j]], add=True)
        return 0

    lax.fori_loop(0, NCHUNK, body, 0)
    plsc.subcore_barrier()
    sl = pl.ds(s * RPT, RPT)

    @pl.when(c == 0)
    def _():
        pltpu.sync_copy(acc.at[sl], out0_hbm.at[sl])

    @pl.when(c == 1)
    def _():
        pltpu.sync_copy(acc.at[sl], out1_hbm.at[sl])


@functools.cache
def _deg_kernel():
    return pl.kernel(
        _deg_body, mesh=_mesh(),
        out_type=(jax.ShapeDtypeStruct((NPAD,), jnp.float32),
                  jax.ShapeDtypeStruct((NPAD,), jnp.float32)),
        scratch_types=[
            pltpu.VMEM((NCHUNK, CK), jnp.int32),
            pltpu.VMEM((NCHUNK, CK), jnp.float32),
            pltpu.VMEM((RPT,), jnp.float32),
            pltpu.VMEM_SHARED((NPAD,), jnp.float32),
        ],
        compiler_params=pltpu.CompilerParams(use_tc_tiling_on_sc=False),
    )


RB = 4
GLEAD = 2


def _agg_body(y3_hbm, src_hbm, dst_hbm, ew16_hbm, out_hbm,
              src_v, dst_v, ew16_v, rows_v, zb_v, acc,
              gsem0, gsem1, gsem2, gsem3, ssem0, ssem1, ssem2, ssem3):
    c = lax.axis_index("c")
    s = lax.axis_index("s")
    gsem = (gsem0, gsem1, gsem2, gsem3)
    ssem = (ssem0, ssem1, ssem2, ssem3)

    def zb(i, _):
        for q in range(FH // 16):
            zb_v[i, pl.ds(q * 16, 16)] = jnp.zeros((16,), jnp.float32)
        return 0

    lax.fori_loop(0, 128, zb, 0)
    for k in range(RPT // 128):
        pltpu.sync_copy(zb_v, acc.at[pl.ds(s * RPT + k * 128, 128)])
    plsc.subcore_barrier()

    pltpu.sync_copy(src_hbm.at[pl.ds(s * NCHUNK2, NCHUNK2)], src_v)
    pltpu.sync_copy(dst_hbm.at[pl.ds(s * NCHUNK2, NCHUNK2)], dst_v)
    yh = y3_hbm.at[c]

    def issue(jn, bn):
        pltpu.async_copy(ew16_hbm.at[s * NCHUNK2 + jn], ew16_v.at[bn],
                         gsem[bn])
        pltpu.async_copy(yh.at[src_v.at[jn]], rows_v.at[bn], gsem[bn])

    def wait_gather(b):
        pltpu.make_async_copy(ew16_hbm.at[0], ew16_v.at[b], gsem[b]).wait()
        pltpu.make_async_copy(yh.at[pl.ds(0, CK)], rows_v.at[b],
                              gsem[b]).wait()

    def drain_scatter(bn):
        pltpu.make_async_copy(rows_v.at[bn], acc.at[pl.ds(0, CK)],
                              ssem[bn]).wait()

    for b in range(GLEAD):
        issue(b, b)

    def outer(j0, _):
        for b in range(RB):
            j = j0 + b
            wait_gather(b)

            def row(r, _):
                wv = ew16_v[b, r, :]
                for q in range(FH // 16):
                    sl = pl.ds(q * 16, 16)
                    rows_v[b, r, sl] = rows_v[b, r, sl] * wv
                return 0

            lax.fori_loop(0, CK, row, 0, unroll=5)
            pltpu.async_copy(rows_v.at[b], acc.at[dst_v.at[j]], ssem[b],
                             add=True)
            jn = j + GLEAD
            bn = (b + GLEAD) % RB

            @pl.when(jn < NCHUNK2)
            def _():
                @pl.when(jn >= RB)
                def _():
                    drain_scatter(bn)

                issue(jn, bn)
        return 0

    lax.fori_loop(0, NCHUNK2 // RB, lambda t, u: outer(t * RB, u), 0)
    for b in range(RB):
        drain_scatter(b)
    plsc.subcore_barrier()
    sl = pl.ds(s * RPT, RPT)
    pltpu.sync_copy(acc.at[sl], out_hbm.at[c, sl])


@functools.cache
def _agg_kernel():
    return pl.kernel(
        _agg_body, mesh=_mesh(),
        out_type=jax.ShapeDtypeStruct((NC, NPAD, FH), jnp.float32),
        scratch_types=[
            pltpu.VMEM((NCHUNK2, CK), jnp.int32),
            pltpu.VMEM((NCHUNK2, CK), jnp.int32),
            pltpu.VMEM((RB, CK, 16), jnp.float32),
            pltpu.VMEM((RB, CK, FH), jnp.float32),
            pltpu.VMEM((128, FH), jnp.float32),
            pltpu.VMEM_SHARED((NPAD, FH), jnp.float32),
            pltpu.SemaphoreType.DMA,
            pltpu.SemaphoreType.DMA,
            pltpu.SemaphoreType.DMA,
            pltpu.SemaphoreType.DMA,
            pltpu.SemaphoreType.DMA,
            pltpu.SemaphoreType.DMA,
            pltpu.SemaphoreType.DMA,
            pltpu.SemaphoreType.DMA,
        ],
        compiler_params=pltpu.CompilerParams(use_tc_tiling_on_sc=False),
    )


EB = NW * NCHUNK
EBLK = 256


def _ew16_body(ew_ref, out_ref):
    k = lax.broadcasted_iota(jnp.int32, (CK, 16 * CK), 0)
    m = lax.broadcasted_iota(jnp.int32, (CK, 16 * CK), 1)
    rep = (m // 16 == k).astype(jnp.float32)
    out_ref[...] = jnp.dot(ew_ref[...], rep,
                           preferred_element_type=jnp.float32)


def _ew16_call(ew):
    return pl.pallas_call(
        _ew16_body,
        grid=(EB // EBLK,),
        in_specs=[pl.BlockSpec((EBLK, CK), lambda i: (i, 0))],
        out_specs=pl.BlockSpec((EBLK, 16 * CK), lambda i: (i, 0)),
        out_shape=jax.ShapeDtypeStruct((EB, 16 * CK), jnp.float32),
    )(ew.reshape(EB, CK))


def _y_body(deg0_ref, deg1_ref, x_ref, y3_ref):
    deg = deg0_ref[...] + deg1_ref[...] + 1.0
    dinv = lax.rsqrt(deg)
    y3_ref[0] = x_ref[:, :FH] * dinv
    y3_ref[1] = x_ref[:, FH:] * dinv


def _y_call(deg0, deg1, x_pad):
    return pl.pallas_call(
        _y_body,
        grid=(GRID,),
        in_specs=[
            pl.BlockSpec((BLK, 1), lambda i: (i, 0)),
            pl.BlockSpec((BLK, 1), lambda i: (i, 0)),
            pl.BlockSpec((BLK, F), lambda i: (i, 0)),
        ],
        out_specs=pl.BlockSpec((NC, BLK, FH), lambda i: (0, i, 0)),
        out_shape=jax.ShapeDtypeStruct((NC, NPAD, FH), jnp.float32),
    )(deg0, deg1, x_pad)


def _head_body(deg0_ref, deg1_ref, agg3_ref, y3_ref, uz_ref,
               uh_ref, bz_ref, bh_ref, wl_ref, bl_ref, o_ref):
    deg = deg0_ref[...] + deg1_ref[...] + 1.0
    dinv = lax.rsqrt(deg)
    xa = jnp.concatenate(
        [agg3_ref[0] + y3_ref[0], agg3_ref[1] + y3_ref[1]], axis=1) * dinv
    z = jax.nn.sigmoid(
        jnp.dot(xa, uz_ref[...], preferred_element_type=jnp.float32)
        + bz_ref[...])
    ht = jnp.tanh(
        jnp.dot(xa, uh_ref[...], preferred_element_type=jnp.float32)
        + bh_ref[...])
    h = jnp.maximum((1.0 - z) * ht, 0.0)
    logits = jnp.dot(h, wl_ref[...], preferred_element_type=jnp.float32) \
        + bl_ref[...]
    m = jnp.max(logits, axis=1, keepdims=True)
    ex = jnp.exp(logits - m)
    o_ref[...] = ex / jnp.sum(ex, axis=1, keepdims=True)


def _head_call(deg0, deg1, agg3, y3, Uz, Uh, cbz, cbh, Wlp, blp):
    return pl.pallas_call(
        _head_body,
        grid=(GRID,),
        in_specs=[
            pl.BlockSpec((BLK, 1), lambda i: (i, 0)),
            pl.BlockSpec((BLK, 1), lambda i: (i, 0)),
            pl.BlockSpec((NC, BLK, FH), lambda i: (0, i, 0)),
            pl.BlockSpec((NC, BLK, FH), lambda i: (0, i, 0)),
            pl.BlockSpec((F, F), lambda i: (0, 0)),
            pl.BlockSpec((F, F), lambda i: (0, 0)),
            pl.BlockSpec((1, F), lambda i: (0, 0)),
            pl.BlockSpec((1, F), lambda i: (0, 0)),
            pl.BlockSpec((F, F), lambda i: (0, 0)),
            pl.BlockSpec((1, F), lambda i: (0, 0)),
        ],
        out_specs=pl.BlockSpec((BLK, F), lambda i: (i, 0)),
        out_shape=jax.ShapeDtypeStruct((NPAD, F), jnp.float32),
    )(deg0, deg1, agg3, y3, Uz, Uh, cbz, cbh, Wlp, blp)


def kernel(x, edge_index, edge_weight, Wz, bz, Wr, br, Wh, bh,
           Wlz, blz, Wlr, blr, Wlh, blh, att, Wlin, blin):
    src = edge_index[0].reshape(NW * NCHUNK, CK)
    dst = edge_index[1].reshape(NW * NCHUNK, CK)
    ew = edge_weight.reshape(NW * NCHUNK, CK)
    ew16 = _ew16_call(edge_weight).reshape(NW * NCHUNK, CK, 16)
    x_pad = jnp.pad(x, ((0, NPAD - N), (0, 0)))

    Uz = Wz @ Wlz[:F]
    cbz = (bz @ Wlz[:F] + blz).reshape(1, F)
    Uh = Wh @ Wlh[:F]
    cbh = (bh @ Wlh[:F] + blh).reshape(1, F)
    Wlp = jnp.zeros((F, F), jnp.float32).at[:, :C].set(Wlin)
    blp = jnp.full((F,), -1e30, jnp.float32).at[:C].set(blin).reshape(1, F)

    deg0, deg1 = _deg_kernel()(dst, ew)
    deg0 = deg0.reshape(NPAD, 1)
    deg1 = deg1.reshape(NPAD, 1)
    y3 = _y_call(deg0, deg1, x_pad)
    agg3 = _agg_kernel()(y3, src, dst, ew16)
    out = _head_call(deg0, deg1, agg3, y3, Uz, Uh, cbz, cbh, Wlp, blp)
    return out[:N, :C]


# speedup vs baseline: 1.2468x; 1.2468x over previous
import functools
import jax
import jax.numpy as jnp
from jax import lax
from jax.experimental import pallas as pl
from jax.experimental.pallas import tpu as pltpu
from jax.experimental.pallas import tpu_sc as plsc

N = 10000
NPAD = 10240
E = 320000
F = 128
C = 10
NC = 2
NS = 16
NW = NC * NS
EPT = E // NW
CK = 125
NCHUNK = EPT // CK
EPT2 = E // NS
NCHUNK2 = EPT2 // CK
FH = F // NC
RPT = NPAD // NS
BLK = 2048
GRID = NPAD // BLK


def _mesh():
    return plsc.VectorSubcoreMesh(core_axis_name="c", subcore_axis_name="s")


def _deg_body(dst_hbm, ew_hbm, out0_hbm, out1_hbm, dst_v, ew_v, zb_v, acc):
    c = lax.axis_index("c")
    s = lax.axis_index("s")
    wid = s * NC + c

    def zb(i, _):
        zb_v[pl.ds(i * 16, 16)] = jnp.zeros((16,), jnp.float32)
        return 0

    lax.fori_loop(0, RPT // 16, zb, 0)
    pltpu.sync_copy(zb_v, acc.at[pl.ds(s * RPT, RPT)])
    plsc.subcore_barrier()

    pltpu.sync_copy(dst_hbm.at[pl.ds(wid * NCHUNK, NCHUNK)], dst_v)
    pltpu.sync_copy(ew_hbm.at[pl.ds(wid * NCHUNK, NCHUNK)], ew_v)

    def body(j, _):
        pltpu.sync_copy(ew_v.at[j], acc.at[dst_v.at[j]], add=True)
        return 0

    lax.fori_loop(0, NCHUNK, body, 0)
    plsc.subcore_barrier()
    sl = pl.ds(s * RPT, RPT)

    @pl.when(c == 0)
    def _():
        pltpu.sync_copy(acc.at[sl], out0_hbm.at[sl])

    @pl.when(c == 1)
    def _():
        pltpu.sync_copy(acc.at[sl], out1_hbm.at[sl])


@functools.cache
def _deg_kernel():
    return pl.kernel(
        _deg_body, mesh=_mesh(),
        out_type=(jax.ShapeDtypeStruct((NPAD,), jnp.float32),
                  jax.ShapeDtypeStruct((NPAD,), jnp.float32)),
        scratch_types=[
            pltpu.VMEM((NCHUNK, CK), jnp.int32),
            pltpu.VMEM((NCHUNK, CK), jnp.float32),
            pltpu.VMEM((RPT,), jnp.float32),
            pltpu.VMEM_SHARED((NPAD,), jnp.float32),
        ],
        compiler_params=pltpu.CompilerParams(use_tc_tiling_on_sc=False),
    )


RB = 4
GLEAD = 2


def _agg_body(y3_hbm, src_hbm, dst_hbm, ew16_hbm, out_hbm,
              src_v, dst_v, ew16_v, rows_v, zb_v, acc,
              gsem0, gsem1, gsem2, gsem3, ssem0, ssem1, ssem2, ssem3):
    c = lax.axis_index("c")
    s = lax.axis_index("s")
    gsem = (gsem0, gsem1, gsem2, gsem3)
    ssem = (ssem0, ssem1, ssem2, ssem3)

    def zb(i, _):
        for q in range(FH // 16):
            zb_v[i, pl.ds(q * 16, 16)] = jnp.zeros((16,), jnp.float32)
        return 0

    lax.fori_loop(0, 128, zb, 0)
    for k in range(RPT // 128):
        pltpu.sync_copy(zb_v, acc.at[pl.ds(s * RPT + k * 128, 128)])
    plsc.subcore_barrier()

    pltpu.sync_copy(src_hbm.at[pl.ds(s * NCHUNK2, NCHUNK2)], src_v)
    pltpu.sync_copy(dst_hbm.at[pl.ds(s * NCHUNK2, NCHUNK2)], dst_v)
    yh = y3_hbm.at[c]

    def issue(jn, bn):
        pltpu.async_copy(ew16_hbm.at[s * NCHUNK2 + jn], ew16_v.at[bn],
                         gsem[bn])
        pltpu.async_copy(yh.at[src_v.at[jn]], rows_v.at[bn], gsem[bn])

    def wait_gather(b):
        pltpu.make_async_copy(ew16_hbm.at[0], ew16_v.at[b], gsem[b]).wait()
        pltpu.make_async_copy(yh.at[pl.ds(0, CK)], rows_v.at[b],
                              gsem[b]).wait()

    def drain_scatter(bn):
        pltpu.make_async_copy(rows_v.at[bn], acc.at[pl.ds(0, CK)],
                              ssem[bn]).wait()

    for b in range(GLEAD):
        issue(b, b)

    def outer(j0, _):
        for b in range(RB):
            j = j0 + b
            wait_gather(b)

            def row(r, _):
                wv = ew16_v[b, r, :]
                for q in range(FH // 16):
                    sl = pl.ds(q * 16, 16)
                    rows_v[b, r, sl] = rows_v[b, r, sl] * wv
                return 0

            lax.fori_loop(0, CK, row, 0, unroll=5)
            pltpu.async_copy(rows_v.at[b], acc.at[dst_v.at[j]], ssem[b],
                             add=True)
            jn = j + GLEAD
            bn = (b + GLEAD) % RB

            @pl.when(jn < NCHUNK2)
            def _():
                @pl.when(jn >= RB)
                def _():
                    drain_scatter(bn)

                issue(jn, bn)
        return 0

    lax.fori_loop(0, NCHUNK2 // RB, lambda t, u: outer(t * RB, u), 0)
    for b in range(RB):
        drain_scatter(b)
    plsc.subcore_barrier()
    sl = pl.ds(s * RPT, RPT)
    pltpu.sync_copy(acc.at[sl], out_hbm.at[c, sl])


@functools.cache
def _agg_kernel():
    return pl.kernel(
        _agg_body, mesh=_mesh(),
        out_type=jax.ShapeDtypeStruct((NC, NPAD, FH), jnp.float32),
        scratch_types=[
            pltpu.VMEM((NCHUNK2, CK), jnp.int32),
            pltpu.VMEM((NCHUNK2, CK), jnp.int32),
            pltpu.VMEM((RB, CK, 16), jnp.float32),
            pltpu.VMEM((RB, CK, FH), jnp.float32),
            pltpu.VMEM((128, FH), jnp.float32),
            pltpu.VMEM_SHARED((NPAD, FH), jnp.float32),
            pltpu.SemaphoreType.DMA,
            pltpu.SemaphoreType.DMA,
            pltpu.SemaphoreType.DMA,
            pltpu.SemaphoreType.DMA,
            pltpu.SemaphoreType.DMA,
            pltpu.SemaphoreType.DMA,
            pltpu.SemaphoreType.DMA,
            pltpu.SemaphoreType.DMA,
        ],
        compiler_params=pltpu.CompilerParams(use_tc_tiling_on_sc=False),
    )


EWL = 160
EB = E // EWL
EBLK = 200


def _ew16_body(ew_ref, out_ref):
    k = lax.broadcasted_iota(jnp.int32, (EWL, 16 * EWL), 0)
    m = lax.broadcasted_iota(jnp.int32, (EWL, 16 * EWL), 1)
    rep = (m // 16 == k).astype(jnp.float32)
    out_ref[...] = jnp.dot(ew_ref[...], rep,
                           preferred_element_type=jnp.float32)


def _ew16_call(ew):
    return pl.pallas_call(
        _ew16_body,
        grid=(EB // EBLK,),
        in_specs=[pl.BlockSpec((EBLK, EWL), lambda i: (i, 0))],
        out_specs=pl.BlockSpec((EBLK, 16 * EWL), lambda i: (i, 0)),
        out_shape=jax.ShapeDtypeStruct((EB, 16 * EWL), jnp.float32),
    )(ew.reshape(EB, EWL))


def _y_body(deg0_ref, deg1_ref, x_ref, y3_ref):
    deg = deg0_ref[...] + deg1_ref[...] + 1.0
    dinv = lax.rsqrt(deg)
    y3_ref[0] = x_ref[:, :FH] * dinv
    y3_ref[1] = x_ref[:, FH:] * dinv


def _y_call(deg0, deg1, x_pad):
    return pl.pallas_call(
        _y_body,
        grid=(GRID,),
        in_specs=[
            pl.BlockSpec((BLK, 1), lambda i: (i, 0)),
            pl.BlockSpec((BLK, 1), lambda i: (i, 0)),
            pl.BlockSpec((BLK, F), lambda i: (i, 0)),
        ],
        out_specs=pl.BlockSpec((NC, BLK, FH), lambda i: (0, i, 0)),
        out_shape=jax.ShapeDtypeStruct((NC, NPAD, FH), jnp.float32),
    )(deg0, deg1, x_pad)


def _head_body(deg0_ref, deg1_ref, agg3_ref, y3_ref, uz_ref,
               uh_ref, bz_ref, bh_ref, wl_ref, bl_ref, o_ref):
    deg = deg0_ref[...] + deg1_ref[...] + 1.0
    dinv = lax.rsqrt(deg)
    xa = jnp.concatenate(
        [agg3_ref[0] + y3_ref[0], agg3_ref[1] + y3_ref[1]], axis=1) * dinv
    z = jax.nn.sigmoid(
        jnp.dot(xa, uz_ref[...], preferred_element_type=jnp.float32)
        + bz_ref[...])
    ht = jnp.tanh(
        jnp.dot(xa, uh_ref[...], preferred_element_type=jnp.float32)
        + bh_ref[...])
    h = jnp.maximum((1.0 - z) * ht, 0.0)
    logits = jnp.dot(h, wl_ref[...], preferred_element_type=jnp.float32) \
        + bl_ref[...]
    m = jnp.max(logits, axis=1, keepdims=True)
    ex = jnp.exp(logits - m)
    o_ref[...] = ex / jnp.sum(ex, axis=1, keepdims=True)


def _head_call(deg0, deg1, agg3, y3, Uz, Uh, cbz, cbh, Wlp, blp):
    return pl.pallas_call(
        _head_body,
        grid=(GRID,),
        in_specs=[
            pl.BlockSpec((BLK, 1), lambda i: (i, 0)),
            pl.BlockSpec((BLK, 1), lambda i: (i, 0)),
            pl.BlockSpec((NC, BLK, FH), lambda i: (0, i, 0)),
            pl.BlockSpec((NC, BLK, FH), lambda i: (0, i, 0)),
            pl.BlockSpec((F, F), lambda i: (0, 0)),
            pl.BlockSpec((F, F), lambda i: (0, 0)),
            pl.BlockSpec((1, F), lambda i: (0, 0)),
            pl.BlockSpec((1, F), lambda i: (0, 0)),
            pl.BlockSpec((F, F), lambda i: (0, 0)),
            pl.BlockSpec((1, F), lambda i: (0, 0)),
        ],
        out_specs=pl.BlockSpec((BLK, F), lambda i: (i, 0)),
        out_shape=jax.ShapeDtypeStruct((NPAD, F), jnp.float32),
    )(deg0, deg1, agg3, y3, Uz, Uh, cbz, cbh, Wlp, blp)


def kernel(x, edge_index, edge_weight, Wz, bz, Wr, br, Wh, bh,
           Wlz, blz, Wlr, blr, Wlh, blh, att, Wlin, blin):
    src = edge_index[0].reshape(NW * NCHUNK, CK)
    dst = edge_index[1].reshape(NW * NCHUNK, CK)
    ew = edge_weight.reshape(NW * NCHUNK, CK)
    ew16 = _ew16_call(edge_weight).reshape(NW * NCHUNK, CK, 16)
    x_pad = jnp.pad(x, ((0, NPAD - N), (0, 0)))

    Uz = Wz @ Wlz[:F]
    cbz = (bz @ Wlz[:F] + blz).reshape(1, F)
    Uh = Wh @ Wlh[:F]
    cbh = (bh @ Wlh[:F] + blh).reshape(1, F)
    Wlp = jnp.zeros((F, F), jnp.float32).at[:, :C].set(Wlin)
    blp = jnp.full((F,), -1e30, jnp.float32).at[:C].set(blin).reshape(1, F)

    deg0, deg1 = _deg_kernel()(dst, ew)
    deg0 = deg0.reshape(NPAD, 1)
    deg1 = deg1.reshape(NPAD, 1)
    y3 = _y_call(deg0, deg1, x_pad)
    agg3 = _agg_kernel()(y3, src, dst, ew16)
    out = _head_call(deg0, deg1, agg3, y3, Uz, Uh, cbz, cbh, Wlp, blp)
    return out[:N, :C]
